# Initial kernel scaffold; baseline (speedup 1.0000x reference)
import jax
import jax.numpy as jnp
from jax.experimental import pallas as pl


def kernel(x, edge_index, W, b):
    raise NotImplementedError("write your pallas kernel here")


# speedup vs baseline: 96.3816x; 96.3816x over previous
import functools

import jax
import jax.numpy as jnp
from jax import lax
from jax.experimental import pallas as pl
from jax.experimental.pallas import tpu as pltpu
from jax.experimental.pallas import tpu_sc as plsc

NW = 32
LANES = 16


def _hist_body(npad, ept, dst_hbm, out_hbm, idx_v, acc_v):
    c = lax.axis_index("c")
    s = lax.axis_index("s")
    wid = c * 16 + s

    zeros = jnp.zeros((LANES,), jnp.float32)

    def zero_body(i, _):
        acc_v[pl.ds(i * LANES, LANES)] = zeros
        return _

    lax.fori_loop(0, npad // LANES, zero_body, 0)

    pltpu.sync_copy(dst_hbm.at[pl.ds(wid * ept, ept)], idx_v)

    ones = jnp.ones((LANES,), jnp.float32)

    def body(j, _):
        idx = idx_v[pl.ds(j * LANES, LANES)]
        plsc.addupdate_scatter(acc_v, [idx], ones)
        return _

    lax.fori_loop(0, ept // LANES, body, 0)

    pltpu.sync_copy(acc_v, out_hbm.at[wid])


def _sc_histogram(dst, npad):
    e = dst.shape[0]
    ept = e // NW
    mesh = plsc.VectorSubcoreMesh(core_axis_name="c", subcore_axis_name="s")
    return pl.kernel(
        functools.partial(_hist_body, npad, ept),
        out_type=jax.ShapeDtypeStruct((NW, npad), jnp.float32),
        mesh=mesh,
        compiler_params=pltpu.CompilerParams(needs_layout_passes=False),
        scratch_types=[
            pltpu.VMEM((ept,), jnp.int32),
            pltpu.VMEM((npad,), jnp.float32),
        ],
    )(dst)


def _msg_body(npad, ept, src_hbm, dst_hbm, y_hbm, out_hbm, src_v, dst_v, y_v, acc_v):
    c = lax.axis_index("c")
    s = lax.axis_index("s")
    wid = c * 16 + s

    zeros = jnp.zeros((LANES,), jnp.float32)

    def zero_body(i, _):
        acc_v[pl.ds(i * LANES, LANES)] = zeros
        return _

    lax.fori_loop(0, npad // LANES, zero_body, 0)

    pltpu.sync_copy(y_hbm, y_v)
    pltpu.sync_copy(src_hbm.at[pl.ds(wid * ept, ept)], src_v)
    pltpu.sync_copy(dst_hbm.at[pl.ds(wid * ept, ept)], dst_v)

    def body(j, _):
        si = src_v[pl.ds(j * LANES, LANES)]
        di = dst_v[pl.ds(j * LANES, LANES)]
        vals = plsc.load_gather(y_v, [si])
        plsc.addupdate_scatter(acc_v, [di], vals)
        return _

    lax.fori_loop(0, ept // LANES, body, 0)

    pltpu.sync_copy(acc_v, out_hbm.at[wid])


def _sc_message(src, dst, y, npad):
    e = src.shape[0]
    ept = e // NW
    mesh = plsc.VectorSubcoreMesh(core_axis_name="c", subcore_axis_name="s")
    return pl.kernel(
        functools.partial(_msg_body, npad, ept),
        out_type=jax.ShapeDtypeStruct((NW, npad), jnp.float32),
        mesh=mesh,
        compiler_params=pltpu.CompilerParams(needs_layout_passes=False),
        scratch_types=[
            pltpu.VMEM((ept,), jnp.int32),
            pltpu.VMEM((ept,), jnp.int32),
            pltpu.VMEM((npad,), jnp.float32),
            pltpu.VMEM((npad,), jnp.float32),
        ],
    )(src, dst, y)


def _d1_body(x3_ref, w_ref, degp_ref, b_ref, y_ref, dis_ref, z_ref):
    xw = jnp.sum(x3_ref[...] * w_ref[...], axis=2)
    deg = jnp.sum(degp_ref[...], axis=0) + 1.0
    dis = lax.rsqrt(deg)
    y = xw * dis
    z = dis * y + b_ref[0, 0]
    y_ref[...] = y
    dis_ref[...] = dis
    z_ref[...] = z


def _tc_dense(x3, w3, degp3, b2, npad):
    rows = npad // 128
    rb = 8
    grid = rows // rb
    vec = jax.ShapeDtypeStruct((rows, 128), jnp.float32)
    return pl.pallas_call(
        _d1_body,
        grid=(grid,),
        in_specs=[
            pl.BlockSpec((rb, 128, 128), lambda i: (i, 0, 0)),
            pl.BlockSpec((1, 1, 128), lambda i: (0, 0, 0)),
            pl.BlockSpec((NW, rb, 128), lambda i: (0, i, 0)),
            pl.BlockSpec((1, 1), lambda i: (0, 0)),
        ],
        out_specs=[
            pl.BlockSpec((rb, 128), lambda i: (i, 0)),
            pl.BlockSpec((rb, 128), lambda i: (i, 0)),
            pl.BlockSpec((rb, 128), lambda i: (i, 0)),
        ],
        out_shape=[vec, vec, vec],
    )(x3, w3, degp3, b2)


def _d2_body(accp_ref, dis_ref, z_ref, out_ref):
    acc = jnp.sum(accp_ref[...], axis=0)
    out_ref[...] = dis_ref[...] * acc + z_ref[...]


def _tc_combine(accp3, dis, z, npad):
    rows = npad // 128
    rb = 8
    grid = rows // rb
    return pl.pallas_call(
        _d2_body,
        grid=(grid,),
        in_specs=[
            pl.BlockSpec((NW, rb, 128), lambda i: (0, i, 0)),
            pl.BlockSpec((rb, 128), lambda i: (i, 0)),
            pl.BlockSpec((rb, 128), lambda i: (i, 0)),
        ],
        out_specs=pl.BlockSpec((rb, 128), lambda i: (i, 0)),
        out_shape=jax.ShapeDtypeStruct((rows, 128), jnp.float32),
    )(accp3, dis, z)


def kernel(x, edge_index, W, b):
    n, d = x.shape
    e = edge_index.shape[1]
    npad = ((n + 1023) // 1024) * 1024
    rows = npad // 128

    src = edge_index[0].astype(jnp.int32)
    dst = edge_index[1].astype(jnp.int32)

    degp = _sc_histogram(dst, npad)

    x_pad = jnp.pad(x, ((0, npad - n), (0, 0)))
    x3 = x_pad.reshape(rows, 128, d)
    w3 = W.reshape(1, 1, d)
    degp3 = degp.reshape(NW, rows, 128)
    b2 = b.reshape(1, 1)
    y2, dis2, z2 = _tc_dense(x3, w3, degp3, b2, npad)

    y_flat = y2.reshape(npad)
    accp = _sc_message(src, dst, y_flat, npad)

    accp3 = accp.reshape(NW, rows, 128)
    out2 = _tc_combine(accp3, dis2, z2, npad)

    return out2.reshape(npad, 1)[:n]


# speedup vs baseline: 111.2743x; 1.1545x over previous
import functools

import jax
import jax.numpy as jnp
from jax import lax
from jax.experimental import pallas as pl
from jax.experimental.pallas import tpu as pltpu
from jax.experimental.pallas import tpu_sc as plsc

NW = 32
LANES = 16


def _hist_body(npad, ept, dst_hbm, out_hbm, idx_v, acc_v):
    c = lax.axis_index("c")
    s = lax.axis_index("s")
    wid = c * 16 + s

    zeros = jnp.zeros((LANES,), jnp.float32)

    @plsc.parallel_loop(0, npad, step=LANES, unroll=8)
    def _zero(i):
        acc_v[pl.ds(i, LANES)] = zeros

    pltpu.sync_copy(dst_hbm.at[pl.ds(wid * ept, ept)], idx_v)

    ones = jnp.ones((LANES,), jnp.float32)

    @plsc.parallel_loop(0, ept, step=LANES, unroll=8)
    def _hist(j):
        idx = idx_v[pl.ds(j, LANES)]
        plsc.addupdate_scatter(acc_v, [idx], ones)

    pltpu.sync_copy(acc_v, out_hbm.at[wid])


def _sc_histogram(dst, npad):
    e = dst.shape[0]
    ept = e // NW
    mesh = plsc.VectorSubcoreMesh(core_axis_name="c", subcore_axis_name="s")
    return pl.kernel(
        functools.partial(_hist_body, npad, ept),
        out_type=jax.ShapeDtypeStruct((NW, npad), jnp.float32),
        mesh=mesh,
        compiler_params=pltpu.CompilerParams(needs_layout_passes=False),
        scratch_types=[
            pltpu.VMEM((ept,), jnp.int32),
            pltpu.VMEM((npad,), jnp.float32),
        ],
    )(dst)


def _msg_body(npad, ept, src_hbm, dst_hbm, y_hbm, out_hbm, src_v, dst_v, y_v, acc_v):
    c = lax.axis_index("c")
    s = lax.axis_index("s")
    wid = c * 16 + s

    zeros = jnp.zeros((LANES,), jnp.float32)

    @plsc.parallel_loop(0, npad, step=LANES, unroll=8)
    def _zero(i):
        acc_v[pl.ds(i, LANES)] = zeros

    pltpu.sync_copy(y_hbm, y_v)
    pltpu.sync_copy(src_hbm.at[pl.ds(wid * ept, ept)], src_v)
    pltpu.sync_copy(dst_hbm.at[pl.ds(wid * ept, ept)], dst_v)

    @plsc.parallel_loop(0, ept, step=LANES, unroll=8)
    def _msg(j):
        si = src_v[pl.ds(j, LANES)]
        di = dst_v[pl.ds(j, LANES)]
        vals = plsc.load_gather(y_v, [si])
        plsc.addupdate_scatter(acc_v, [di], vals)

    pltpu.sync_copy(acc_v, out_hbm.at[wid])


def _sc_message(src, dst, y, npad):
    e = src.shape[0]
    ept = e // NW
    mesh = plsc.VectorSubcoreMesh(core_axis_name="c", subcore_axis_name="s")
    return pl.kernel(
        functools.partial(_msg_body, npad, ept),
        out_type=jax.ShapeDtypeStruct((NW, npad), jnp.float32),
        mesh=mesh,
        compiler_params=pltpu.CompilerParams(needs_layout_passes=False),
        scratch_types=[
            pltpu.VMEM((ept,), jnp.int32),
            pltpu.VMEM((ept,), jnp.int32),
            pltpu.VMEM((npad,), jnp.float32),
            pltpu.VMEM((npad,), jnp.float32),
        ],
    )(src, dst, y)


def _d1_body(x3_ref, w_ref, degp_ref, b_ref, y_ref, dis_ref, z_ref):
    xw = jnp.sum(x3_ref[...] * w_ref[...], axis=2)
    deg = jnp.sum(degp_ref[...], axis=0) + 1.0
    dis = lax.rsqrt(deg)
    y = xw * dis
    z = dis * y + b_ref[0, 0]
    y_ref[...] = y
    dis_ref[...] = dis
    z_ref[...] = z


def _tc_dense(x3, w3, degp3, b2, npad):
    rows = npad // 128
    rb = 8
    grid = rows // rb
    vec = jax.ShapeDtypeStruct((rows, 128), jnp.float32)
    return pl.pallas_call(
        _d1_body,
        grid=(grid,),
        in_specs=[
            pl.BlockSpec((rb, 128, 128), lambda i: (i, 0, 0)),
            pl.BlockSpec((1, 1, 128), lambda i: (0, 0, 0)),
            pl.BlockSpec((NW, rb, 128), lambda i: (0, i, 0)),
            pl.BlockSpec((1, 1), lambda i: (0, 0)),
        ],
        out_specs=[
            pl.BlockSpec((rb, 128), lambda i: (i, 0)),
            pl.BlockSpec((rb, 128), lambda i: (i, 0)),
            pl.BlockSpec((rb, 128), lambda i: (i, 0)),
        ],
        out_shape=[vec, vec, vec],
    )(x3, w3, degp3, b2)


def _d2_body(accp_ref, dis_ref, z_ref, out_ref):
    acc = jnp.sum(accp_ref[...], axis=0)
    out_ref[...] = dis_ref[...] * acc + z_ref[...]


def _tc_combine(accp3, dis, z, npad):
    rows = npad // 128
    rb = 8
    grid = rows // rb
    return pl.pallas_call(
        _d2_body,
        grid=(grid,),
        in_specs=[
            pl.BlockSpec((NW, rb, 128), lambda i: (0, i, 0)),
            pl.BlockSpec((rb, 128), lambda i: (i, 0)),
            pl.BlockSpec((rb, 128), lambda i: (i, 0)),
        ],
        out_specs=pl.BlockSpec((rb, 128), lambda i: (i, 0)),
        out_shape=jax.ShapeDtypeStruct((rows, 128), jnp.float32),
    )(accp3, dis, z)


def kernel(x, edge_index, W, b):
    n, d = x.shape
    e = edge_index.shape[1]
    npad = ((n + 1023) // 1024) * 1024
    rows = npad // 128

    src = edge_index[0].astype(jnp.int32)
    dst = edge_index[1].astype(jnp.int32)

    degp = _sc_histogram(dst, npad)

    x_pad = jnp.pad(x, ((0, npad - n), (0, 0)))
    x3 = x_pad.reshape(rows, 128, d)
    w3 = W.reshape(1, 1, d)
    degp3 = degp.reshape(NW, rows, 128)
    b2 = b.reshape(1, 1)
    y2, dis2, z2 = _tc_dense(x3, w3, degp3, b2, npad)

    y_flat = y2.reshape(npad)
    accp = _sc_message(src, dst, y_flat, npad)

    accp3 = accp.reshape(NW, rows, 128)
    out2 = _tc_combine(accp3, dis2, z2, npad)

    return out2.reshape(npad, 1)[:n]
